# baseline (speedup 1.0000x reference)
import jax
import jax.numpy as jnp
import numpy as np
from jax import lax
from jax.experimental import pallas as pl
from jax.experimental.pallas import tpu as pltpu

_N = 20000
_NP = 20480
_ROWS = _NP // 128
_NSEL = 1000
_NDET = 100
_OUT_ROWS = 104
_NMS_THRESH = 0.5
_SCORE_THRESH = 0.05
_CLIP = float(np.log(1000.0 / 16.0))
_IMG_H, _IMG_W = 600.0, 600.0


def _body(inp_ref, out_ref, sc_ref, b0_ref, b1_ref, b2_ref, b3_ref,
          k0_ref, k1_ref, k2_ref, k3_ref, kv_ref,
          s0_ref, s1_ref, s2_ref, s3_ref):
    x1 = inp_ref[0]
    y1 = inp_ref[1]
    x2 = inp_ref[2]
    y2 = inp_ref[3]
    w = x2 - x1
    h = y2 - y1
    cx = x1 + 0.5 * w
    cy = y1 + 0.5 * h
    dx = inp_ref[4] / 10.0
    dy = inp_ref[5] / 10.0
    dw = jnp.minimum(inp_ref[6] / 5.0, _CLIP)
    dh = jnp.minimum(inp_ref[7] / 5.0, _CLIP)
    pcx = dx * w + cx
    pcy = dy * h + cy
    pw = jnp.exp(dw) * w
    ph = jnp.exp(dh) * h
    b0_ref[...] = jnp.clip(pcx - 0.5 * pw, 0.0, _IMG_W)
    b1_ref[...] = jnp.clip(pcy - 0.5 * ph, 0.0, _IMG_H)
    b2_ref[...] = jnp.clip(pcx + 0.5 * pw, 0.0, _IMG_W)
    b3_ref[...] = jnp.clip(pcy + 0.5 * ph, 0.0, _IMG_H)

    l0 = inp_ref[8]
    l1 = inp_ref[9]
    lm = jnp.maximum(l0, l1)
    e0 = jnp.exp(l0 - lm)
    e1 = jnp.exp(l1 - lm)
    s = e1 / (e0 + e1)
    s = jnp.where(s > _SCORE_THRESH, s, 0.0)
    flat = (lax.broadcasted_iota(jnp.int32, (_ROWS, 128), 0) * 128
            + lax.broadcasted_iota(jnp.int32, (_ROWS, 128), 1))
    s = jnp.where(flat < _N, s, -1.0)
    sc_ref[...] = s

    nvreg = _ROWS // 8
    lane128 = lax.broadcasted_iota(jnp.int32, (1, 128), 1)
    lidx = (lax.broadcasted_iota(jnp.int32, (8, 128), 0) * 128
            + lax.broadcasted_iota(jnp.int32, (8, 128), 1))
    big = jnp.int32(1 << 30)
    tm = jnp.full((1, 128), -3.0, jnp.float32)
    tg = jnp.full((1, 128), big, jnp.int32)
    for v in range(nvreg):
        blk = s[8 * v:8 * v + 8, :]
        mv = jnp.max(blk)
        gv = jnp.min(jnp.where(blk == mv, lidx, big)) + v * 1024
        tm = jnp.where(lane128 == v, mv, tm)
        tg = jnp.where(lane128 == v, gv, tg)

    kv_ref[...] = jnp.zeros((8, 128), jnp.float32)
    s0_ref[...] = jnp.zeros((8, 128), jnp.float32)
    s1_ref[...] = jnp.zeros((8, 128), jnp.float32)
    s2_ref[...] = jnp.zeros((8, 128), jnp.float32)
    s3_ref[...] = jnp.zeros((8, 128), jnp.float32)

    lanes = lax.broadcasted_iota(jnp.int32, (1, 128), 1)
    kidx = (lax.broadcasted_iota(jnp.int32, (8, 128), 0) * 128
            + lax.broadcasted_iota(jnp.int32, (8, 128), 1))
    lane5 = lax.broadcasted_iota(jnp.int32, (1, 5), 1)

    def body(r, carry):
        kc, tm, tg = carry
        m = jnp.max(tm)
        g = jnp.min(jnp.where(tm == m, tg, big))
        t = g // 1024
        gl = g - t * 1024
        st = sc_ref[pl.ds(t * 8, 8), :]
        onehot = lidx == gl
        newst = jnp.where(onehot, -2.0, st)
        sc_ref[pl.ds(t * 8, 8), :] = newst
        ntm = jnp.max(newst)
        ntg = jnp.min(jnp.where(newst == ntm, lidx, big)) + t * 1024
        tm = jnp.where(lane128 == t, ntm, tm)
        tg = jnp.where(lane128 == t, ntg, tg)

        bx1 = jnp.sum(jnp.where(onehot, b0_ref[pl.ds(t * 8, 8), :], 0.0))
        by1 = jnp.sum(jnp.where(onehot, b1_ref[pl.ds(t * 8, 8), :], 0.0))
        bx2 = jnp.sum(jnp.where(onehot, b2_ref[pl.ds(t * 8, 8), :], 0.0))
        by2 = jnp.sum(jnp.where(onehot, b3_ref[pl.ds(t * 8, 8), :], 0.0))

        kx1 = k0_ref[...]
        ky1 = k1_ref[...]
        kx2 = k2_ref[...]
        ky2 = k3_ref[...]
        kv = kv_ref[...]
        area_a = (kx2 - kx1) * (ky2 - ky1)
        area_b = (bx2 - bx1) * (by2 - by1)
        ltx = jnp.maximum(kx1, bx1)
        lty = jnp.maximum(ky1, by1)
        rbx = jnp.minimum(kx2, bx2)
        rby = jnp.minimum(ky2, by2)
        iw = jnp.clip(rbx - ltx, 0.0, None)
        ih = jnp.clip(rby - lty, 0.0, None)
        inter = iw * ih
        iou = inter / (area_a + area_b - inter + 1e-9)
        sup = (iou > _NMS_THRESH) & (kv > 0.5)
        nsup = jnp.max(jnp.where(sup, 1.0, 0.0))
        keep = nsup == 0.0

        at_k = (kidx == kc) & keep
        k0_ref[...] = jnp.where(at_k, bx1, kx1)
        k1_ref[...] = jnp.where(at_k, by1, ky1)
        k2_ref[...] = jnp.where(at_k, bx2, kx2)
        k3_ref[...] = jnp.where(at_k, by2, ky2)
        kv_ref[...] = jnp.where(at_k, 1.0, kv)

        p = jnp.where(keep & (kc < _NDET), kc, _NDET)
        row = jnp.where(lane5 == 0, bx1,
              jnp.where(lane5 == 1, by1,
              jnp.where(lane5 == 2, bx2,
              jnp.where(lane5 == 3, by2, m))))
        out_ref[pl.ds(p, 1), :] = row

        sq = r - kc
        at_s = (kidx == sq) & (~keep)
        s0_ref[...] = jnp.where(at_s, bx1, s0_ref[...])
        s1_ref[...] = jnp.where(at_s, by1, s1_ref[...])
        s2_ref[...] = jnp.where(at_s, bx2, s2_ref[...])
        s3_ref[...] = jnp.where(at_s, by2, s3_ref[...])

        return kc + jnp.where(keep, 1, 0), tm, tg

    kc, _, _ = lax.fori_loop(0, _NSEL, body, (jnp.int32(0), tm, tg))

    def fill(j, _):
        p = kc + j
        valid = p < _NDET
        onehot = lanes == j
        sx1 = jnp.sum(jnp.where(onehot, s0_ref[pl.ds(0, 1), :], 0.0))
        sy1 = jnp.sum(jnp.where(onehot, s1_ref[pl.ds(0, 1), :], 0.0))
        sx2 = jnp.sum(jnp.where(onehot, s2_ref[pl.ds(0, 1), :], 0.0))
        sy2 = jnp.sum(jnp.where(onehot, s3_ref[pl.ds(0, 1), :], 0.0))
        row = jnp.where(lane5 == 0, sx1,
              jnp.where(lane5 == 1, sy1,
              jnp.where(lane5 == 2, sx2,
              jnp.where(lane5 == 3, sy2, -1.0))))
        pw = jnp.where(valid, p, _NDET)
        out_ref[pl.ds(pw, 1), :] = row
        return 0

    lax.fori_loop(0, _NDET, fill, 0)


def kernel(proposals, box_regression, logits):
    pad = _NP - _N
    P = jnp.pad(proposals.astype(jnp.float32), ((0, pad), (0, 0)))
    R = jnp.pad(box_regression.astype(jnp.float32), ((0, pad), (0, 0)))
    L = jnp.pad(logits.astype(jnp.float32), ((0, pad), (0, 0)))
    stk = jnp.concatenate([P, R, L], axis=1)
    inp = stk.T.reshape(10, _ROWS, 128)

    out = pl.pallas_call(
        _body,
        out_shape=jax.ShapeDtypeStruct((_OUT_ROWS, 5), jnp.float32),
        scratch_shapes=[
            pltpu.VMEM((_ROWS, 128), jnp.float32),
            pltpu.VMEM((_ROWS, 128), jnp.float32),
            pltpu.VMEM((_ROWS, 128), jnp.float32),
            pltpu.VMEM((_ROWS, 128), jnp.float32),
            pltpu.VMEM((_ROWS, 128), jnp.float32),
            pltpu.VMEM((8, 128), jnp.float32),
            pltpu.VMEM((8, 128), jnp.float32),
            pltpu.VMEM((8, 128), jnp.float32),
            pltpu.VMEM((8, 128), jnp.float32),
            pltpu.VMEM((8, 128), jnp.float32),
            pltpu.VMEM((8, 128), jnp.float32),
            pltpu.VMEM((8, 128), jnp.float32),
            pltpu.VMEM((8, 128), jnp.float32),
            pltpu.VMEM((8, 128), jnp.float32),
        ],
    )(inp)
    return out[:_NDET]


# speedup vs baseline: 11.4508x; 11.4508x over previous
import jax
import jax.numpy as jnp
import numpy as np
from jax import lax
from jax.experimental import pallas as pl
from jax.experimental.pallas import tpu as pltpu

_N = 20000
_NP = 20480
_ROWS = _NP // 128
_NSEL = 1000
_NDET = 100
_OUT_ROWS = 104
_NMS_THRESH = 0.5
_SCORE_THRESH = 0.05
_CLIP = float(np.log(1000.0 / 16.0))
_IMG_H, _IMG_W = 600.0, 600.0


def _body(inp_ref, out_ref, sc_ref, b0_ref, b1_ref, b2_ref, b3_ref,
          k0_ref, k1_ref, k2_ref, k3_ref, kv_ref,
          s0_ref, s1_ref, s2_ref, s3_ref):
    x1 = inp_ref[0]
    y1 = inp_ref[1]
    x2 = inp_ref[2]
    y2 = inp_ref[3]
    w = x2 - x1
    h = y2 - y1
    cx = x1 + 0.5 * w
    cy = y1 + 0.5 * h
    dx = inp_ref[4] / 10.0
    dy = inp_ref[5] / 10.0
    dw = jnp.minimum(inp_ref[6] / 5.0, _CLIP)
    dh = jnp.minimum(inp_ref[7] / 5.0, _CLIP)
    pcx = dx * w + cx
    pcy = dy * h + cy
    pw = jnp.exp(dw) * w
    ph = jnp.exp(dh) * h
    b0_ref[...] = jnp.clip(pcx - 0.5 * pw, 0.0, _IMG_W)
    b1_ref[...] = jnp.clip(pcy - 0.5 * ph, 0.0, _IMG_H)
    b2_ref[...] = jnp.clip(pcx + 0.5 * pw, 0.0, _IMG_W)
    b3_ref[...] = jnp.clip(pcy + 0.5 * ph, 0.0, _IMG_H)

    l0 = inp_ref[8]
    l1 = inp_ref[9]
    lm = jnp.maximum(l0, l1)
    e0 = jnp.exp(l0 - lm)
    e1 = jnp.exp(l1 - lm)
    s = e1 / (e0 + e1)
    s = jnp.where(s > _SCORE_THRESH, s, 0.0)
    flat = (lax.broadcasted_iota(jnp.int32, (_ROWS, 128), 0) * 128
            + lax.broadcasted_iota(jnp.int32, (_ROWS, 128), 1))
    s = jnp.where(flat < _N, s, -1.0)
    sc_ref[...] = s

    kv_ref[...] = jnp.zeros((8, 128), jnp.float32)
    s0_ref[...] = jnp.zeros((8, 128), jnp.float32)
    s1_ref[...] = jnp.zeros((8, 128), jnp.float32)
    s2_ref[...] = jnp.zeros((8, 128), jnp.float32)
    s3_ref[...] = jnp.zeros((8, 128), jnp.float32)

    lanes = lax.broadcasted_iota(jnp.int32, (1, 128), 1)
    kidx = (lax.broadcasted_iota(jnp.int32, (8, 128), 0) * 128
            + lax.broadcasted_iota(jnp.int32, (8, 128), 1))
    lane5 = lax.broadcasted_iota(jnp.int32, (1, 5), 1)

    def body(carry):
        r, kc = carry
        s = sc_ref[...]
        m = jnp.max(s)
        idxm = jnp.min(jnp.where(s == m, flat, jnp.int32(1 << 30)))
        sub = idxm // 128
        lane = idxm % 128
        onehot = lanes == lane
        row_s = sc_ref[pl.ds(sub, 1), :]
        sc_ref[pl.ds(sub, 1), :] = jnp.where(onehot, -2.0, row_s)

        bx1 = jnp.sum(jnp.where(onehot, b0_ref[pl.ds(sub, 1), :], 0.0))
        by1 = jnp.sum(jnp.where(onehot, b1_ref[pl.ds(sub, 1), :], 0.0))
        bx2 = jnp.sum(jnp.where(onehot, b2_ref[pl.ds(sub, 1), :], 0.0))
        by2 = jnp.sum(jnp.where(onehot, b3_ref[pl.ds(sub, 1), :], 0.0))

        kx1 = k0_ref[...]
        ky1 = k1_ref[...]
        kx2 = k2_ref[...]
        ky2 = k3_ref[...]
        kv = kv_ref[...]
        area_a = (kx2 - kx1) * (ky2 - ky1)
        area_b = (bx2 - bx1) * (by2 - by1)
        ltx = jnp.maximum(kx1, bx1)
        lty = jnp.maximum(ky1, by1)
        rbx = jnp.minimum(kx2, bx2)
        rby = jnp.minimum(ky2, by2)
        iw = jnp.clip(rbx - ltx, 0.0, None)
        ih = jnp.clip(rby - lty, 0.0, None)
        inter = iw * ih
        iou = inter / (area_a + area_b - inter + 1e-9)
        sup = (iou > _NMS_THRESH) & (kv > 0.5)
        nsup = jnp.max(jnp.where(sup, 1.0, 0.0))
        keep = nsup == 0.0

        at_k = (kidx == kc) & keep
        k0_ref[...] = jnp.where(at_k, bx1, kx1)
        k1_ref[...] = jnp.where(at_k, by1, ky1)
        k2_ref[...] = jnp.where(at_k, bx2, kx2)
        k3_ref[...] = jnp.where(at_k, by2, ky2)
        kv_ref[...] = jnp.where(at_k, 1.0, kv)

        p = jnp.where(keep & (kc < _NDET), kc, _NDET)
        row = jnp.where(lane5 == 0, bx1,
              jnp.where(lane5 == 1, by1,
              jnp.where(lane5 == 2, bx2,
              jnp.where(lane5 == 3, by2, m))))
        out_ref[pl.ds(p, 1), :] = row

        sq = r - kc
        at_s = (kidx == sq) & (~keep)
        s0_ref[...] = jnp.where(at_s, bx1, s0_ref[...])
        s1_ref[...] = jnp.where(at_s, by1, s1_ref[...])
        s2_ref[...] = jnp.where(at_s, bx2, s2_ref[...])
        s3_ref[...] = jnp.where(at_s, by2, s3_ref[...])

        return r + 1, kc + jnp.where(keep, 1, 0)

    _, kc = lax.while_loop(
        lambda c: (c[0] < _NSEL) & (c[1] < _NDET),
        body, (jnp.int32(0), jnp.int32(0)))

    def fill(j):
        p = kc + j
        valid = p < _NDET
        onehot = lanes == j
        sx1 = jnp.sum(jnp.where(onehot, s0_ref[pl.ds(0, 1), :], 0.0))
        sy1 = jnp.sum(jnp.where(onehot, s1_ref[pl.ds(0, 1), :], 0.0))
        sx2 = jnp.sum(jnp.where(onehot, s2_ref[pl.ds(0, 1), :], 0.0))
        sy2 = jnp.sum(jnp.where(onehot, s3_ref[pl.ds(0, 1), :], 0.0))
        row = jnp.where(lane5 == 0, sx1,
              jnp.where(lane5 == 1, sy1,
              jnp.where(lane5 == 2, sx2,
              jnp.where(lane5 == 3, sy2, -1.0))))
        pw = jnp.where(valid, p, _NDET)
        out_ref[pl.ds(pw, 1), :] = row
        return j + 1

    lax.while_loop(lambda j: j < _NDET - kc, fill, jnp.int32(0))


def kernel(proposals, box_regression, logits):
    pad = _NP - _N
    P = jnp.pad(proposals.astype(jnp.float32), ((0, pad), (0, 0)))
    R = jnp.pad(box_regression.astype(jnp.float32), ((0, pad), (0, 0)))
    L = jnp.pad(logits.astype(jnp.float32), ((0, pad), (0, 0)))
    stk = jnp.concatenate([P, R, L], axis=1)
    inp = stk.T.reshape(10, _ROWS, 128)

    out = pl.pallas_call(
        _body,
        out_shape=jax.ShapeDtypeStruct((_OUT_ROWS, 5), jnp.float32),
        scratch_shapes=[
            pltpu.VMEM((_ROWS, 128), jnp.float32),
            pltpu.VMEM((_ROWS, 128), jnp.float32),
            pltpu.VMEM((_ROWS, 128), jnp.float32),
            pltpu.VMEM((_ROWS, 128), jnp.float32),
            pltpu.VMEM((_ROWS, 128), jnp.float32),
            pltpu.VMEM((8, 128), jnp.float32),
            pltpu.VMEM((8, 128), jnp.float32),
            pltpu.VMEM((8, 128), jnp.float32),
            pltpu.VMEM((8, 128), jnp.float32),
            pltpu.VMEM((8, 128), jnp.float32),
            pltpu.VMEM((8, 128), jnp.float32),
            pltpu.VMEM((8, 128), jnp.float32),
            pltpu.VMEM((8, 128), jnp.float32),
            pltpu.VMEM((8, 128), jnp.float32),
        ],
    )(inp)
    return out[:_NDET]


# speedup vs baseline: 14.1425x; 1.2351x over previous
import jax
import jax.numpy as jnp
import numpy as np
from jax import lax
from jax.experimental import pallas as pl
from jax.experimental.pallas import tpu as pltpu

_N = 20000
_NP = 20480
_ROWS = _NP // 128
_NSEL = 1000
_NDET = 100
_OUT_ROWS = 104
_NMS_THRESH = 0.5
_SCORE_THRESH = 0.05
_CLIP = float(np.log(1000.0 / 16.0))
_IMG_H, _IMG_W = 600.0, 600.0


def _body(inp_ref, out_ref, sc_ref, b0_ref, b1_ref, b2_ref, b3_ref,
          k0_ref, k1_ref, k2_ref, k3_ref, kv_ref,
          s0_ref, s1_ref, s2_ref, s3_ref):
    x1 = inp_ref[0]
    y1 = inp_ref[1]
    x2 = inp_ref[2]
    y2 = inp_ref[3]
    w = x2 - x1
    h = y2 - y1
    cx = x1 + 0.5 * w
    cy = y1 + 0.5 * h
    dx = inp_ref[4] / 10.0
    dy = inp_ref[5] / 10.0
    dw = jnp.minimum(inp_ref[6] / 5.0, _CLIP)
    dh = jnp.minimum(inp_ref[7] / 5.0, _CLIP)
    pcx = dx * w + cx
    pcy = dy * h + cy
    pw = jnp.exp(dw) * w
    ph = jnp.exp(dh) * h
    b0_ref[...] = jnp.clip(pcx - 0.5 * pw, 0.0, _IMG_W)
    b1_ref[...] = jnp.clip(pcy - 0.5 * ph, 0.0, _IMG_H)
    b2_ref[...] = jnp.clip(pcx + 0.5 * pw, 0.0, _IMG_W)
    b3_ref[...] = jnp.clip(pcy + 0.5 * ph, 0.0, _IMG_H)

    l0 = inp_ref[8]
    l1 = inp_ref[9]
    lm = jnp.maximum(l0, l1)
    e0 = jnp.exp(l0 - lm)
    e1 = jnp.exp(l1 - lm)
    s = e1 / (e0 + e1)
    s = jnp.where(s > _SCORE_THRESH, s, 0.0)
    flat = (lax.broadcasted_iota(jnp.int32, (_ROWS, 128), 0) * 128
            + lax.broadcasted_iota(jnp.int32, (_ROWS, 128), 1))
    s = jnp.where(flat < _N, s, -1.0)
    sc_ref[...] = s

    kv_ref[...] = jnp.zeros((8, 128), jnp.float32)
    s0_ref[...] = jnp.zeros((8, 128), jnp.float32)
    s1_ref[...] = jnp.zeros((8, 128), jnp.float32)
    s2_ref[...] = jnp.zeros((8, 128), jnp.float32)
    s3_ref[...] = jnp.zeros((8, 128), jnp.float32)

    lanes = lax.broadcasted_iota(jnp.int32, (1, 128), 1)
    kidx = (lax.broadcasted_iota(jnp.int32, (8, 128), 0) * 128
            + lax.broadcasted_iota(jnp.int32, (8, 128), 1))
    lane5 = lax.broadcasted_iota(jnp.int32, (1, 5), 1)

    def extract():
        s = sc_ref[...]
        m = jnp.max(s)
        idxm = jnp.min(jnp.where(s == m, flat, jnp.int32(1 << 30)))
        sub = idxm // 128
        lane = idxm % 128
        onehot = lanes == lane
        row_s = sc_ref[pl.ds(sub, 1), :]
        sc_ref[pl.ds(sub, 1), :] = jnp.where(onehot, -2.0, row_s)
        bx1 = jnp.sum(jnp.where(onehot, b0_ref[pl.ds(sub, 1), :], 0.0))
        by1 = jnp.sum(jnp.where(onehot, b1_ref[pl.ds(sub, 1), :], 0.0))
        bx2 = jnp.sum(jnp.where(onehot, b2_ref[pl.ds(sub, 1), :], 0.0))
        by2 = jnp.sum(jnp.where(onehot, b3_ref[pl.ds(sub, 1), :], 0.0))
        return m, bx1, by1, bx2, by2

    def body(carry):
        r, kc, m, bx1, by1, bx2, by2 = carry
        nxt = extract()

        kx1 = k0_ref[...]
        ky1 = k1_ref[...]
        kx2 = k2_ref[...]
        ky2 = k3_ref[...]
        kv = kv_ref[...]
        area_a = (kx2 - kx1) * (ky2 - ky1)
        area_b = (bx2 - bx1) * (by2 - by1)
        ltx = jnp.maximum(kx1, bx1)
        lty = jnp.maximum(ky1, by1)
        rbx = jnp.minimum(kx2, bx2)
        rby = jnp.minimum(ky2, by2)
        iw = jnp.clip(rbx - ltx, 0.0, None)
        ih = jnp.clip(rby - lty, 0.0, None)
        inter = iw * ih
        iou = inter / (area_a + area_b - inter + 1e-9)
        sup = (iou > _NMS_THRESH) & (kv > 0.5)
        nsup = jnp.max(jnp.where(sup, 1.0, 0.0))
        keep = nsup == 0.0

        at_k = (kidx == kc) & keep
        k0_ref[...] = jnp.where(at_k, bx1, kx1)
        k1_ref[...] = jnp.where(at_k, by1, ky1)
        k2_ref[...] = jnp.where(at_k, bx2, kx2)
        k3_ref[...] = jnp.where(at_k, by2, ky2)
        kv_ref[...] = jnp.where(at_k, 1.0, kv)

        p = jnp.where(keep & (kc < _NDET), kc, _NDET)
        row = jnp.where(lane5 == 0, bx1,
              jnp.where(lane5 == 1, by1,
              jnp.where(lane5 == 2, bx2,
              jnp.where(lane5 == 3, by2, m))))
        out_ref[pl.ds(p, 1), :] = row

        sq = r - kc
        at_s = (kidx == sq) & (~keep)
        s0_ref[...] = jnp.where(at_s, bx1, s0_ref[...])
        s1_ref[...] = jnp.where(at_s, by1, s1_ref[...])
        s2_ref[...] = jnp.where(at_s, bx2, s2_ref[...])
        s3_ref[...] = jnp.where(at_s, by2, s3_ref[...])

        return (r + 1, kc + jnp.where(keep, 1, 0)) + nxt

    fin = lax.while_loop(
        lambda c: (c[0] < _NSEL) & (c[1] < _NDET),
        body, (jnp.int32(0), jnp.int32(0)) + extract())
    kc = fin[1]

    def fill(j):
        p = kc + j
        valid = p < _NDET
        onehot = lanes == j
        sx1 = jnp.sum(jnp.where(onehot, s0_ref[pl.ds(0, 1), :], 0.0))
        sy1 = jnp.sum(jnp.where(onehot, s1_ref[pl.ds(0, 1), :], 0.0))
        sx2 = jnp.sum(jnp.where(onehot, s2_ref[pl.ds(0, 1), :], 0.0))
        sy2 = jnp.sum(jnp.where(onehot, s3_ref[pl.ds(0, 1), :], 0.0))
        row = jnp.where(lane5 == 0, sx1,
              jnp.where(lane5 == 1, sy1,
              jnp.where(lane5 == 2, sx2,
              jnp.where(lane5 == 3, sy2, -1.0))))
        pw = jnp.where(valid, p, _NDET)
        out_ref[pl.ds(pw, 1), :] = row
        return j + 1

    lax.while_loop(lambda j: j < _NDET - kc, fill, jnp.int32(0))


def kernel(proposals, box_regression, logits):
    pad = _NP - _N
    P = jnp.pad(proposals.astype(jnp.float32), ((0, pad), (0, 0)))
    R = jnp.pad(box_regression.astype(jnp.float32), ((0, pad), (0, 0)))
    L = jnp.pad(logits.astype(jnp.float32), ((0, pad), (0, 0)))
    stk = jnp.concatenate([P, R, L], axis=1)
    inp = stk.T.reshape(10, _ROWS, 128)

    out = pl.pallas_call(
        _body,
        out_shape=jax.ShapeDtypeStruct((_OUT_ROWS, 5), jnp.float32),
        scratch_shapes=[
            pltpu.VMEM((_ROWS, 128), jnp.float32),
            pltpu.VMEM((_ROWS, 128), jnp.float32),
            pltpu.VMEM((_ROWS, 128), jnp.float32),
            pltpu.VMEM((_ROWS, 128), jnp.float32),
            pltpu.VMEM((_ROWS, 128), jnp.float32),
            pltpu.VMEM((8, 128), jnp.float32),
            pltpu.VMEM((8, 128), jnp.float32),
            pltpu.VMEM((8, 128), jnp.float32),
            pltpu.VMEM((8, 128), jnp.float32),
            pltpu.VMEM((8, 128), jnp.float32),
            pltpu.VMEM((8, 128), jnp.float32),
            pltpu.VMEM((8, 128), jnp.float32),
            pltpu.VMEM((8, 128), jnp.float32),
            pltpu.VMEM((8, 128), jnp.float32),
        ],
    )(inp)
    return out[:_NDET]
